# baseline (speedup 1.0000x reference)
import functools

import jax
import jax.numpy as jnp
from jax import lax
from jax.experimental import pallas as pl
from jax.experimental.pallas import tpu as pltpu
from jax.experimental.pallas import tpu_sc as plsc

B, S, HID = 4, 8192, 128
N = B * S

NC, NS, L = 2, 16, 16
NW = NC * NS
NPW = N // NW
WPB = S // NPW
C = 64
NCHUNK = NPW // C
NB = 8
P = 4

_mesh = plsc.VectorSubcoreMesh(core_axis_name="c", subcore_axis_name="s")


@functools.partial(
    pl.kernel,
    mesh=_mesh,
    out_type=jax.ShapeDtypeStruct((B, S, HID), jnp.float32),
    scratch_types=(
        [pltpu.VMEM((NPW,), jnp.int32)] * 2
        + [pltpu.VMEM((C, HID), jnp.float32)] * NB
        + [pltpu.SemaphoreType.DMA] * (2 * NB + 1)
    ),
)
def _embed_add(wids_hbm, pids_hbm, wtab_hbm, ptab_hbm, out_hbm,
               widx_v, pidx_v, *rest):
    bufs = rest[:NB]
    semw = rest[NB:2 * NB]
    sema = rest[2 * NB:3 * NB]
    so = rest[3 * NB]
    wid = lax.axis_index("s") * NC + lax.axis_index("c")
    row = wid // WPB
    off = (wid % WPB) * NPW
    pltpu.sync_copy(wids_hbm.at[row, pl.ds(off, NPW)], widx_v)
    pltpu.sync_copy(pids_hbm.at[row, pl.ds(off, NPW)], pidx_v)

    def fire_w(c):
        return pltpu.async_copy(wtab_hbm.at[widx_v.at[pl.ds(c * C, C)]],
                                bufs[c % NB], semw[c % NB])

    def fire_p(c):
        return pltpu.async_copy(ptab_hbm.at[pidx_v.at[pl.ds(c * C, C)]],
                                bufs[c % NB], sema[c % NB], add=True)

    def fire_out(c):
        return pltpu.async_copy(bufs[c % NB],
                                out_hbm.at[row, pl.ds(off + c * C, C)], so)

    gw = [None] * NCHUNK
    gp = [None] * NCHUNK
    outs = [None] * NCHUNK
    for c in range(min(P, NCHUNK)):
        gw[c] = fire_w(c)
    for c in range(NCHUNK):
        gw[c].wait()
        gp[c] = fire_p(c)
        if c + P < NCHUNK:
            if c + P >= NB:
                outs[c + P - NB].wait()
            gw[c + P] = fire_w(c + P)
        gp[c].wait()
        outs[c] = fire_out(c)
    for c in range(max(0, NCHUNK - NB), NCHUNK):
        outs[c].wait()


def kernel(input_ids, position_ids, word_embeddings, position_embeddings):
    return _embed_add(input_ids.astype(jnp.int32),
                      position_ids.astype(jnp.int32),
                      word_embeddings, position_embeddings)


# speedup vs baseline: 1.1113x; 1.1113x over previous
import functools

import jax
import jax.numpy as jnp
from jax import lax
from jax.experimental import pallas as pl
from jax.experimental.pallas import tpu as pltpu
from jax.experimental.pallas import tpu_sc as plsc

B, S, HID = 4, 8192, 128
N = B * S

NC, NS, L = 2, 16, 16
NW = NC * NS
NPW = N // NW
WPB = S // NPW
C = 128
NCHUNK = NPW // C
NB = 6
P = 3

_mesh = plsc.VectorSubcoreMesh(core_axis_name="c", subcore_axis_name="s")


@functools.partial(
    pl.kernel,
    mesh=_mesh,
    out_type=jax.ShapeDtypeStruct((B, S, HID), jnp.float32),
    scratch_types=(
        [pltpu.VMEM((NPW,), jnp.int32)] * 2
        + [pltpu.VMEM((C, HID), jnp.float32)] * NB
        + [pltpu.SemaphoreType.DMA] * (2 * NB + 1)
    ),
)
def _embed_add(wids_hbm, pids_hbm, wtab_hbm, ptab_hbm, out_hbm,
               widx_v, pidx_v, *rest):
    bufs = rest[:NB]
    semw = rest[NB:2 * NB]
    sema = rest[2 * NB:3 * NB]
    so = rest[3 * NB]
    wid = lax.axis_index("s") * NC + lax.axis_index("c")
    row = wid // WPB
    off = (wid % WPB) * NPW
    pltpu.sync_copy(wids_hbm.at[row, pl.ds(off, NPW)], widx_v)
    pltpu.sync_copy(pids_hbm.at[row, pl.ds(off, NPW)], pidx_v)

    def fire_w(c):
        return pltpu.async_copy(wtab_hbm.at[widx_v.at[pl.ds(c * C, C)]],
                                bufs[c % NB], semw[c % NB])

    def fire_p(c):
        return pltpu.async_copy(ptab_hbm.at[pidx_v.at[pl.ds(c * C, C)]],
                                bufs[c % NB], sema[c % NB], add=True)

    def fire_out(c):
        return pltpu.async_copy(bufs[c % NB],
                                out_hbm.at[row, pl.ds(off + c * C, C)], so)

    gw = [None] * NCHUNK
    gp = [None] * NCHUNK
    outs = [None] * NCHUNK
    for c in range(min(P, NCHUNK)):
        gw[c] = fire_w(c)
    for c in range(NCHUNK):
        gw[c].wait()
        gp[c] = fire_p(c)
        if c + P < NCHUNK:
            if c + P >= NB:
                outs[c + P - NB].wait()
            gw[c + P] = fire_w(c + P)
        gp[c].wait()
        outs[c] = fire_out(c)
    for c in range(max(0, NCHUNK - NB), NCHUNK):
        outs[c].wait()


def kernel(input_ids, position_ids, word_embeddings, position_embeddings):
    return _embed_add(input_ids.astype(jnp.int32),
                      position_ids.astype(jnp.int32),
                      word_embeddings, position_embeddings)


# speedup vs baseline: 1.2320x; 1.1086x over previous
import functools

import jax
import jax.numpy as jnp
from jax import lax
from jax.experimental import pallas as pl
from jax.experimental.pallas import tpu as pltpu
from jax.experimental.pallas import tpu_sc as plsc

B, S, HID = 4, 8192, 128
N = B * S
MAX_TOK = 8192

NC, NS, L = 2, 16, 16
NW = NC * NS
NPW = N // NW
WPB = S // NPW
C = 128
NCHUNK = NPW // C
NB = 3
P = 2
STG = MAX_TOK // NS

_mesh = plsc.VectorSubcoreMesh(core_axis_name="c", subcore_axis_name="s")


@functools.partial(
    pl.kernel,
    mesh=_mesh,
    out_type=jax.ShapeDtypeStruct((B, S, HID), jnp.float32),
    scratch_types=(
        [pltpu.VMEM((NPW,), jnp.int32)] * 2
        + [pltpu.VMEM((C, HID), jnp.float32)] * NB
        + [pltpu.VMEM_SHARED((MAX_TOK, HID), jnp.float32)]
        + [pltpu.SemaphoreType.DMA] * (2 * NB + 2)
    ),
)
def _embed_add(wids_hbm, pids_hbm, wtab_hbm, ptab_hbm, out_hbm,
               widx_v, pidx_v, *rest):
    bufs = rest[:NB]
    ptab_sh = rest[NB]
    semw = rest[NB + 1:2 * NB + 1]
    sema = rest[2 * NB + 1:3 * NB + 1]
    so = rest[3 * NB + 1]
    sst = rest[3 * NB + 2]
    cid = lax.axis_index("c")
    sid = lax.axis_index("s")
    wid = sid * NC + cid
    row = wid // WPB
    off = (wid % WPB) * NPW
    stg = pltpu.async_copy(ptab_hbm.at[pl.ds(sid * STG, STG)],
                           ptab_sh.at[pl.ds(sid * STG, STG)], sst)
    pltpu.sync_copy(wids_hbm.at[row, pl.ds(off, NPW)], widx_v)
    pltpu.sync_copy(pids_hbm.at[row, pl.ds(off, NPW)], pidx_v)

    def fire_w(c):
        return pltpu.async_copy(wtab_hbm.at[widx_v.at[pl.ds(c * C, C)]],
                                bufs[c % NB], semw[c % NB])

    def fire_p(c):
        return pltpu.async_copy(ptab_sh.at[pidx_v.at[pl.ds(c * C, C)]],
                                bufs[c % NB], sema[c % NB], add=True)

    def fire_out(c):
        return pltpu.async_copy(bufs[c % NB],
                                out_hbm.at[row, pl.ds(off + c * C, C)], so)

    gw = [None] * NCHUNK
    gp = [None] * NCHUNK
    outs = [None] * NCHUNK
    for c in range(min(P, NCHUNK)):
        gw[c] = fire_w(c)
    stg.wait()
    plsc.subcore_barrier()
    for c in range(NCHUNK):
        gw[c].wait()
        gp[c] = fire_p(c)
        if c + P < NCHUNK:
            if c + P >= NB:
                outs[c + P - NB].wait()
            gw[c + P] = fire_w(c + P)
        gp[c].wait()
        outs[c] = fire_out(c)
    for c in range(max(0, NCHUNK - NB), NCHUNK):
        outs[c].wait()


def kernel(input_ids, position_ids, word_embeddings, position_embeddings):
    return _embed_add(input_ids.astype(jnp.int32),
                      position_ids.astype(jnp.int32),
                      word_embeddings, position_embeddings)


# speedup vs baseline: 1.2368x; 1.0039x over previous
import functools

import jax
import jax.numpy as jnp
from jax import lax
from jax.experimental import pallas as pl
from jax.experimental.pallas import tpu as pltpu
from jax.experimental.pallas import tpu_sc as plsc

B, S, HID = 4, 8192, 128
N = B * S
MAX_TOK = 8192

NC, NS, L = 2, 16, 16
NW = NC * NS
NPW = N // NW
WPB = S // NPW
C = 128
NCHUNK = NPW // C
NB = 3
P = 2
STG = MAX_TOK // NS

_mesh = plsc.VectorSubcoreMesh(core_axis_name="c", subcore_axis_name="s")


@functools.partial(
    pl.kernel,
    mesh=_mesh,
    out_type=jax.ShapeDtypeStruct((B, S, HID), jnp.float32),
    scratch_types=(
        [pltpu.VMEM((NPW,), jnp.int32)] * 2
        + [pltpu.VMEM((C, HID), jnp.float32)] * NB
        + [pltpu.VMEM_SHARED((MAX_TOK, HID), jnp.float32)]
        + [pltpu.SemaphoreType.DMA] * (2 * NB + 2)
    ),
)
def _embed_add(wids_hbm, pids_hbm, wtab_hbm, ptab_hbm, out_hbm,
               widx_v, pidx_v, *rest):
    bufs = rest[:NB]
    ptab_sh = rest[NB]
    semw = rest[NB + 1:2 * NB + 1]
    sema = rest[2 * NB + 1:3 * NB + 1]
    so = rest[3 * NB + 1]
    sst = rest[3 * NB + 2]
    cid = lax.axis_index("c")
    sid = lax.axis_index("s")
    wid = sid * NC + cid
    row = wid // WPB
    off = (wid % WPB) * NPW
    stg = pltpu.async_copy(ptab_hbm.at[pl.ds(sid * STG, STG)],
                           ptab_sh.at[pl.ds(sid * STG, STG)], sst)
    pltpu.sync_copy(wids_hbm.at[row, pl.ds(off, NPW)], widx_v)
    pltpu.sync_copy(pids_hbm.at[row, pl.ds(off, NPW)], pidx_v)

    def fire_w(c):
        return pltpu.async_copy(wtab_hbm.at[widx_v.at[pl.ds(c * C, C)]],
                                bufs[c % NB], semw[c % NB])

    def fire_p(c):
        return pltpu.async_copy(ptab_sh.at[pidx_v.at[pl.ds(c * C, C)]],
                                bufs[c % NB], sema[c % NB], add=True)

    def fire_out(c):
        return pltpu.async_copy(bufs[c % NB],
                                out_hbm.at[row, pl.ds(off + c * C, C)], so)

    gw = [None] * NCHUNK
    gp = [None] * NCHUNK
    outs = [None] * NCHUNK
    for c in range(min(P, NCHUNK)):
        gw[c] = fire_w(c)
    for c in range(NCHUNK):
        gw[c].wait()
        if c == 0:
            stg.wait()
            plsc.subcore_barrier()
        gp[c] = fire_p(c)
        if c + P < NCHUNK:
            if c + P >= NB:
                outs[c + P - NB].wait()
            gw[c + P] = fire_w(c + P)
        gp[c].wait()
        outs[c] = fire_out(c)
    for c in range(max(0, NCHUNK - NB), NCHUNK):
        outs[c].wait()


def kernel(input_ids, position_ids, word_embeddings, position_embeddings):
    return _embed_add(input_ids.astype(jnp.int32),
                      position_ids.astype(jnp.int32),
                      word_embeddings, position_embeddings)
